# Initial kernel scaffold; baseline (speedup 1.0000x reference)
import jax
import jax.numpy as jnp
from jax.experimental import pallas as pl


def kernel(x, knots, coeffs):
    raise NotImplementedError("write your pallas kernel here")


# speedup vs baseline: 3097.1694x; 3097.1694x over previous
import functools

import jax
import jax.numpy as jnp
from jax import lax
from jax.experimental import pallas as pl
from jax.experimental.pallas import tpu as pltpu
from jax.experimental.pallas import tpu_sc as plsc

NKNOTS = 4096
NINT = NKNOTS - 1
INV_H = NINT / 2.0
N_TOTAL = 8388608
NC, NS, L = 2, 16, 16
NW = NC * NS
CHUNK = N_TOTAL // NW
BLK = 16384
NBLK = CHUNK // BLK
VPB = BLK // L


def _spline_body(x_hbm, knots_hbm, coeffs_hbm, out_hbm,
                 y_v, kn_v, h_v, dl_v, d_v, b_v, c_v, e_v, xin, outb):
    wid = lax.axis_index("s") * NC + lax.axis_index("c")
    base = wid * CHUNK
    lanes = lax.iota(jnp.int32, L)

    pltpu.sync_copy(coeffs_hbm, y_v)
    pltpu.sync_copy(knots_hbm, kn_v)

    def pass1(k, carry):
        i0 = k * L
        idx = lanes + i0
        idxp = jnp.minimum(idx + 1, NKNOTS - 1)
        y0 = y_v[pl.ds(i0, L)]
        y1 = plsc.load_gather(y_v, [idxp])
        k0 = kn_v[pl.ds(i0, L)]
        k1 = plsc.load_gather(kn_v, [idxp])
        hh = (k1 - k0) + 1e-12
        h_v[pl.ds(i0, L)] = hh
        dl_v[pl.ds(i0, L)] = (y1 - y0) / hh
        return carry

    lax.fori_loop(0, NKNOTS // L, pass1, 0)

    def pass2(k, carry):
        i0 = k * L
        idx = lanes + i0
        idxm = jnp.maximum(idx - 1, 0)
        dln = dl_v[pl.ds(i0, L)]
        dlp = plsc.load_gather(dl_v, [idxm])
        hn = h_v[pl.ds(i0, L)]
        hp = plsc.load_gather(h_v, [idxm])
        same = (dlp * dln) > 0.0
        w1 = 2.0 * hn + hp
        w2 = hn + 2.0 * hp
        den = w1 / (dlp + 1e-12) + w2 / (dln + 1e-12) + 1e-12
        d = jnp.where(same, (w1 + w2) / den, 0.0)
        d = jnp.where(idx == 0, dln, d)
        d = jnp.where(idx == NKNOTS - 1, dlp, d)
        d_v[pl.ds(i0, L)] = d
        return carry

    lax.fori_loop(0, NKNOTS // L, pass2, 0)

    def pass3(k, carry):
        i0 = k * L
        idx = lanes + i0
        idxp = jnp.minimum(idx + 1, NKNOTS - 1)
        y0 = y_v[pl.ds(i0, L)]
        y1 = plsc.load_gather(y_v, [idxp])
        d0 = d_v[pl.ds(i0, L)]
        d1 = plsc.load_gather(d_v, [idxp])
        hh = h_v[pl.ds(i0, L)]
        b = hh * d0
        hd1 = hh * d1
        dy = y1 - y0
        b_v[pl.ds(i0, L)] = b
        c_v[pl.ds(i0, L)] = 3.0 * dy - 2.0 * b - hd1
        e_v[pl.ds(i0, L)] = -2.0 * dy + b + hd1
        return carry

    lax.fori_loop(0, NKNOTS // L, pass3, 0)

    def blk_body(j, carry):
        off = base + j * BLK
        pltpu.sync_copy(x_hbm.at[pl.ds(off, BLK)], xin)

        def inner(i, icarry):
            xo = i * L
            xv = xin[pl.ds(xo, L)]
            ff = xv * INV_H + INV_H
            ffc = jnp.minimum(jnp.maximum(ff, 0.0), float(NINT - 1))
            fi = ffc.astype(jnp.int32)
            t = ff - fi.astype(jnp.float32)
            a = plsc.load_gather(y_v, [fi])
            b = plsc.load_gather(b_v, [fi])
            c = plsc.load_gather(c_v, [fi])
            e = plsc.load_gather(e_v, [fi])
            tcl = jnp.minimum(jnp.maximum(t, 0.0), 1.0)
            dt = t - tcl
            slope = jnp.where(dt < 0.0, b, b + 2.0 * c + 3.0 * e)
            outb[pl.ds(xo, L)] = (
                a + tcl * (b + tcl * (c + tcl * e)) + slope * dt)
            return icarry

        lax.fori_loop(0, VPB, inner, 0, unroll=4)
        pltpu.sync_copy(outb, out_hbm.at[pl.ds(off, BLK)])
        return carry

    lax.fori_loop(0, NBLK, blk_body, 0)


_spline_call = pl.kernel(
    _spline_body,
    out_type=jax.ShapeDtypeStruct((N_TOTAL,), jnp.float32),
    mesh=plsc.VectorSubcoreMesh(core_axis_name="c", subcore_axis_name="s"),
    compiler_params=pltpu.CompilerParams(needs_layout_passes=False),
    scratch_types=[
        pltpu.VMEM((NKNOTS,), jnp.float32),
        pltpu.VMEM((NKNOTS,), jnp.float32),
        pltpu.VMEM((NKNOTS,), jnp.float32),
        pltpu.VMEM((NKNOTS,), jnp.float32),
        pltpu.VMEM((NKNOTS,), jnp.float32),
        pltpu.VMEM((NKNOTS,), jnp.float32),
        pltpu.VMEM((NKNOTS,), jnp.float32),
        pltpu.VMEM((NKNOTS,), jnp.float32),
        pltpu.VMEM((BLK,), jnp.float32),
        pltpu.VMEM((BLK,), jnp.float32),
    ],
)


def kernel(x, knots, coeffs):
    return _spline_call(x, knots, coeffs)


# speedup vs baseline: 9500.4646x; 3.0675x over previous
import functools

import jax
import jax.numpy as jnp
from jax import lax
from jax.experimental import pallas as pl
from jax.experimental.pallas import tpu as pltpu
from jax.experimental.pallas import tpu_sc as plsc

NKNOTS = 4096
NINT = NKNOTS - 1
INV_H = NINT / 2.0
N_TOTAL = 8388608
NC, NS, L = 2, 16, 16
NW = NC * NS
CHUNK = N_TOTAL // NW
BLK = 16384
NBLK = CHUNK // BLK
VPB = BLK // L


def _spline_body(x_hbm, knots_hbm, coeffs_hbm, out_hbm,
                 y_v, kn_v, h_v, dl_v, d_v, b_v, c_v, e_v,
                 xin0, xin1, outb0, outb1, si0, si1, so0, so1):
    wid = lax.axis_index("s") * NC + lax.axis_index("c")
    base = wid * CHUNK
    lanes = lax.iota(jnp.int32, L)

    pltpu.sync_copy(coeffs_hbm, y_v)
    pltpu.sync_copy(knots_hbm, kn_v)

    @plsc.parallel_loop(0, NKNOTS // L, unroll=4)
    def pass1(k):
        i0 = k * L
        idx = lanes + i0
        idxp = jnp.minimum(idx + 1, NKNOTS - 1)
        y0 = y_v[pl.ds(i0, L)]
        y1 = plsc.load_gather(y_v, [idxp])
        k0 = kn_v[pl.ds(i0, L)]
        k1 = plsc.load_gather(kn_v, [idxp])
        hh = (k1 - k0) + 1e-12
        h_v[pl.ds(i0, L)] = hh
        dl_v[pl.ds(i0, L)] = (y1 - y0) / hh

    @plsc.parallel_loop(0, NKNOTS // L, unroll=4)
    def pass2(k):
        i0 = k * L
        idx = lanes + i0
        idxm = jnp.maximum(idx - 1, 0)
        dln = dl_v[pl.ds(i0, L)]
        dlp = plsc.load_gather(dl_v, [idxm])
        hn = h_v[pl.ds(i0, L)]
        hp = plsc.load_gather(h_v, [idxm])
        same = (dlp * dln) > 0.0
        w1 = 2.0 * hn + hp
        w2 = hn + 2.0 * hp
        den = w1 / (dlp + 1e-12) + w2 / (dln + 1e-12) + 1e-12
        d = jnp.where(same, (w1 + w2) / den, 0.0)
        d = jnp.where(idx == 0, dln, d)
        d = jnp.where(idx == NKNOTS - 1, dlp, d)
        d_v[pl.ds(i0, L)] = d

    @plsc.parallel_loop(0, NKNOTS // L, unroll=4)
    def pass3(k):
        i0 = k * L
        idx = lanes + i0
        idxp = jnp.minimum(idx + 1, NKNOTS - 1)
        y0 = y_v[pl.ds(i0, L)]
        y1 = plsc.load_gather(y_v, [idxp])
        d0 = d_v[pl.ds(i0, L)]
        d1 = plsc.load_gather(d_v, [idxp])
        hh = h_v[pl.ds(i0, L)]
        b = hh * d0
        hd1 = hh * d1
        dy = y1 - y0
        b_v[pl.ds(i0, L)] = b
        c_v[pl.ds(i0, L)] = 3.0 * dy - 2.0 * b - hd1
        e_v[pl.ds(i0, L)] = -2.0 * dy + b + hd1

    xin = (xin0, xin1)
    outb = (outb0, outb1)
    si = (si0, si1)
    so = (so0, so1)

    def compute_block(src_ref, dst_ref):
        @plsc.parallel_loop(0, VPB, unroll=4)
        def inner(i):
            xo = i * L
            xv = src_ref[pl.ds(xo, L)]
            ff = xv * INV_H + INV_H
            ffc = jnp.minimum(jnp.maximum(ff, 0.0), float(NINT - 1))
            fi = ffc.astype(jnp.int32)
            t = ff - fi.astype(jnp.float32)
            a = plsc.load_gather(y_v, [fi])
            b = plsc.load_gather(b_v, [fi])
            c = plsc.load_gather(c_v, [fi])
            e = plsc.load_gather(e_v, [fi])
            tcl = jnp.minimum(jnp.maximum(t, 0.0), 1.0)
            dt = t - tcl
            slope = jnp.where(dt < 0.0, b, b + 2.0 * c + 3.0 * e)
            dst_ref[pl.ds(xo, L)] = (
                a + tcl * (b + tcl * (c + tcl * e)) + slope * dt)

    pend_in = [None, None]
    pend_out = [None, None]
    for j in range(2):
        pend_in[j] = pltpu.async_copy(
            x_hbm.at[pl.ds(base + j * BLK, BLK)], xin[j], si[j])
    for j in range(NBLK):
        p = j % 2
        pend_in[p].wait()
        if pend_out[p] is not None:
            pend_out[p].wait()
        compute_block(xin[p], outb[p])
        pend_out[p] = pltpu.async_copy(
            outb[p], out_hbm.at[pl.ds(base + j * BLK, BLK)], so[p])
        if j + 2 < NBLK:
            pend_in[p] = pltpu.async_copy(
                x_hbm.at[pl.ds(base + (j + 2) * BLK, BLK)], xin[p], si[p])
    pend_out[0].wait()
    pend_out[1].wait()


_spline_call = pl.kernel(
    _spline_body,
    out_type=jax.ShapeDtypeStruct((N_TOTAL,), jnp.float32),
    mesh=plsc.VectorSubcoreMesh(core_axis_name="c", subcore_axis_name="s"),
    compiler_params=pltpu.CompilerParams(needs_layout_passes=False),
    scratch_types=[
        pltpu.VMEM((NKNOTS,), jnp.float32),
        pltpu.VMEM((NKNOTS,), jnp.float32),
        pltpu.VMEM((NKNOTS,), jnp.float32),
        pltpu.VMEM((NKNOTS,), jnp.float32),
        pltpu.VMEM((NKNOTS,), jnp.float32),
        pltpu.VMEM((NKNOTS,), jnp.float32),
        pltpu.VMEM((NKNOTS,), jnp.float32),
        pltpu.VMEM((NKNOTS,), jnp.float32),
        pltpu.VMEM((BLK,), jnp.float32),
        pltpu.VMEM((BLK,), jnp.float32),
        pltpu.VMEM((BLK,), jnp.float32),
        pltpu.VMEM((BLK,), jnp.float32),
        pltpu.SemaphoreType.DMA,
        pltpu.SemaphoreType.DMA,
        pltpu.SemaphoreType.DMA,
        pltpu.SemaphoreType.DMA,
    ],
)


def kernel(x, knots, coeffs):
    return _spline_call(x, knots, coeffs)


# speedup vs baseline: 10400.4788x; 1.0947x over previous
import functools

import jax
import jax.numpy as jnp
from jax import lax
from jax.experimental import pallas as pl
from jax.experimental.pallas import tpu as pltpu
from jax.experimental.pallas import tpu_sc as plsc

NKNOTS = 4096
NINT = NKNOTS - 1
INV_H = NINT / 2.0
N_TOTAL = 8388608
NC, NS, L = 2, 16, 16
NW = NC * NS
CHUNK = N_TOTAL // NW
BLK = 16384
NBLK = CHUNK // BLK
VPB = BLK // L


def _spline_body(x_hbm, knots_hbm, coeffs_hbm, out_hbm,
                 y_v, kn_v, h_v, dl_v, d_v, b_v, c_v, e_v,
                 xin0, xin1, outb0, outb1, si0, si1, so0, so1):
    wid = lax.axis_index("s") * NC + lax.axis_index("c")
    base = wid * CHUNK
    lanes = lax.iota(jnp.int32, L)

    pltpu.sync_copy(coeffs_hbm, y_v)
    pltpu.sync_copy(knots_hbm, kn_v)

    @plsc.parallel_loop(0, NKNOTS // L, unroll=4)
    def pass1(k):
        i0 = k * L
        idx = lanes + i0
        idxp = jnp.minimum(idx + 1, NKNOTS - 1)
        y0 = y_v[pl.ds(i0, L)]
        y1 = plsc.load_gather(y_v, [idxp])
        k0 = kn_v[pl.ds(i0, L)]
        k1 = plsc.load_gather(kn_v, [idxp])
        hh = (k1 - k0) + 1e-12
        h_v[pl.ds(i0, L)] = hh
        dl_v[pl.ds(i0, L)] = (y1 - y0) / hh

    @plsc.parallel_loop(0, NKNOTS // L, unroll=4)
    def pass2(k):
        i0 = k * L
        idx = lanes + i0
        idxm = jnp.maximum(idx - 1, 0)
        dln = dl_v[pl.ds(i0, L)]
        dlp = plsc.load_gather(dl_v, [idxm])
        hn = h_v[pl.ds(i0, L)]
        hp = plsc.load_gather(h_v, [idxm])
        same = (dlp * dln) > 0.0
        w1 = 2.0 * hn + hp
        w2 = hn + 2.0 * hp
        den = w1 / (dlp + 1e-12) + w2 / (dln + 1e-12) + 1e-12
        d = jnp.where(same, (w1 + w2) / den, 0.0)
        d = jnp.where(idx == 0, dln, d)
        d = jnp.where(idx == NKNOTS - 1, dlp, d)
        d_v[pl.ds(i0, L)] = d

    @plsc.parallel_loop(0, NKNOTS // L, unroll=4)
    def pass3(k):
        i0 = k * L
        idx = lanes + i0
        idxp = jnp.minimum(idx + 1, NKNOTS - 1)
        y0 = y_v[pl.ds(i0, L)]
        y1 = plsc.load_gather(y_v, [idxp])
        d0 = d_v[pl.ds(i0, L)]
        d1 = plsc.load_gather(d_v, [idxp])
        hh = h_v[pl.ds(i0, L)]
        b = hh * d0
        hd1 = hh * d1
        dy = y1 - y0
        b_v[pl.ds(i0, L)] = b
        c_v[pl.ds(i0, L)] = 3.0 * dy - 2.0 * b - hd1
        e_v[pl.ds(i0, L)] = -2.0 * dy + b + hd1

    xin = (xin0, xin1)
    outb = (outb0, outb1)
    si = (si0, si1)
    so = (so0, so1)

    def compute_block(src_ref, dst_ref):
        @plsc.parallel_loop(0, VPB, unroll=8)
        def inner(i):
            xo = i * L
            xv = src_ref[pl.ds(xo, L)]
            ff = xv * INV_H + INV_H
            ffc = jnp.minimum(jnp.maximum(ff, 0.0), float(NINT - 1))
            fi = ffc.astype(jnp.int32)
            t = ff - fi.astype(jnp.float32)
            a = plsc.load_gather(y_v, [fi])
            b = plsc.load_gather(b_v, [fi])
            c = plsc.load_gather(c_v, [fi])
            e = plsc.load_gather(e_v, [fi])
            tcl = jnp.minimum(jnp.maximum(t, 0.0), 1.0)
            dt = t - tcl
            slope = jnp.where(dt < 0.0, b, b + 2.0 * c + 3.0 * e)
            dst_ref[pl.ds(xo, L)] = (
                a + tcl * (b + tcl * (c + tcl * e)) + slope * dt)

    pend_in = [None, None]
    pend_out = [None, None]
    for j in range(2):
        pend_in[j] = pltpu.async_copy(
            x_hbm.at[pl.ds(base + j * BLK, BLK)], xin[j], si[j])
    for j in range(NBLK):
        p = j % 2
        pend_in[p].wait()
        if pend_out[p] is not None:
            pend_out[p].wait()
        compute_block(xin[p], outb[p])
        pend_out[p] = pltpu.async_copy(
            outb[p], out_hbm.at[pl.ds(base + j * BLK, BLK)], so[p])
        if j + 2 < NBLK:
            pend_in[p] = pltpu.async_copy(
                x_hbm.at[pl.ds(base + (j + 2) * BLK, BLK)], xin[p], si[p])
    pend_out[0].wait()
    pend_out[1].wait()


_spline_call = pl.kernel(
    _spline_body,
    out_type=jax.ShapeDtypeStruct((N_TOTAL,), jnp.float32),
    mesh=plsc.VectorSubcoreMesh(core_axis_name="c", subcore_axis_name="s"),
    compiler_params=pltpu.CompilerParams(needs_layout_passes=False),
    scratch_types=[
        pltpu.VMEM((NKNOTS,), jnp.float32),
        pltpu.VMEM((NKNOTS,), jnp.float32),
        pltpu.VMEM((NKNOTS,), jnp.float32),
        pltpu.VMEM((NKNOTS,), jnp.float32),
        pltpu.VMEM((NKNOTS,), jnp.float32),
        pltpu.VMEM((NKNOTS,), jnp.float32),
        pltpu.VMEM((NKNOTS,), jnp.float32),
        pltpu.VMEM((NKNOTS,), jnp.float32),
        pltpu.VMEM((BLK,), jnp.float32),
        pltpu.VMEM((BLK,), jnp.float32),
        pltpu.VMEM((BLK,), jnp.float32),
        pltpu.VMEM((BLK,), jnp.float32),
        pltpu.SemaphoreType.DMA,
        pltpu.SemaphoreType.DMA,
        pltpu.SemaphoreType.DMA,
        pltpu.SemaphoreType.DMA,
    ],
)


def kernel(x, knots, coeffs):
    return _spline_call(x, knots, coeffs)
